# baseline (speedup 1.0000x reference)
import functools

import jax
import jax.numpy as jnp
from jax import lax
from jax.experimental import pallas as pl
from jax.experimental.pallas import tpu as pltpu
from jax.experimental.pallas import tpu_sc as plsc

F32 = jnp.float32
HI = jax.lax.Precision.HIGHEST

NP = 10112
RPT = NP // 16
SR = 10016
LAST = 15 * RPT
NT = 32
C = 128
CA = 16
NCH = 40
W = 48
NJ = 6
RB = 1264
JCOL = [(0, 48), (48, 48), (96, 48), (144, 48), (192, 48), (240, 16)]


def _tables_body(z_ref, x_ref, ws_ref, wd_ref, w1_ref, ta_ref, zt_ref,
                 *pq_refs):
    zb = z_ref[...]
    xb = x_ref[...]
    h = ws_ref.shape[1]
    for k in range(2):
        a = jnp.dot(zb, ws_ref[k].T, precision=HI, preferred_element_type=F32)
        u = jnp.dot(a, wd_ref[k], precision=HI, preferred_element_type=F32)
        ta_ref[:, k * 256:(k + 1) * 256] = u
    zt_ref[...] = zb
    p = jnp.dot(xb, w1_ref[:, :h].T, precision=HI, preferred_element_type=F32)
    q = jnp.dot(xb, w1_ref[:, h:].T, precision=HI, preferred_element_type=F32)
    zpad = jnp.zeros((zb.shape[0], W - JCOL[-1][1]), F32)
    for j, (c0, cw) in enumerate(JCOL):
        ps = p[:, c0:c0 + cw]
        qs = q[:, c0:c0 + cw]
        if cw < W:
            ps = jnp.concatenate([ps, zpad], axis=1)
            qs = jnp.concatenate([qs, zpad], axis=1)
        pq_refs[j][...] = ps
        pq_refs[NJ + j][...] = qs


def _build_tables(zGp, xtp, Ws, Wd, F2_w1):
    grid = NP // RB
    wspec = pl.BlockSpec(Ws.shape, lambda i: (0, 0, 0))
    w1spec = pl.BlockSpec(F2_w1.shape, lambda i: (0, 0))
    row = lambda w: pl.BlockSpec((RB, w), lambda i: (i, 0))
    return pl.pallas_call(
        _tables_body,
        grid=(grid,),
        in_specs=[row(256), row(256), wspec, wspec, w1spec],
        out_specs=[row(512), row(256)] + [row(W)] * (2 * NJ),
        out_shape=[jax.ShapeDtypeStruct((NP, 512), F32),
                   jax.ShapeDtypeStruct((NP, 256), F32)]
                  + [jax.ShapeDtypeStruct((NP, W), F32)] * (2 * NJ),
    )(zGp, xtp, Ws, Wd, F2_w1)


def _sc_edges(srci, dsti, ta, zt, pqs, z96, z16):
    mesh = plsc.VectorSubcoreMesh(core_axis_name="c", subcore_axis_name="s",
                                  num_cores=2, num_subcores=16)

    @functools.partial(
        pl.kernel,
        out_type=(jax.ShapeDtypeStruct((2, NJ, NP, 2 * W), F32),
                  jax.ShapeDtypeStruct((2, NP, 16), F32)),
        mesh=mesh,
        compiler_params=pltpu.CompilerParams(use_tc_tiling_on_sc=False),
        scratch_types=dict(
            sidx=pltpu.VMEM((NCH, C), jnp.int32),
            didx=pltpu.VMEM((NCH, C), jnp.int32),
            exb=pltpu.VMEM((NCH * C // 8, 16), F32),
            dnb=pltpu.VMEM((C, 16), F32),
            S=pltpu.VMEM_SHARED((SR, 2 * W), F32),
            DN=pltpu.VMEM_SHARED((SR, 16), F32),
            sg0=pltpu.SemaphoreType.DMA,
            sg1=pltpu.SemaphoreType.DMA,
            sg2=pltpu.SemaphoreType.DMA,
            sg3=pltpu.SemaphoreType.DMA,
            ss0=pltpu.SemaphoreType.DMA,
            ss1=pltpu.SemaphoreType.DMA,
        ),
    )
    def body(srci_h, dsti_h, ta_h, zt_h, p0_h, p1_h, p2_h, p3_h, p4_h, p5_h,
             q0_h, q1_h, q2_h, q3_h, q4_h, q5_h, z96_h, z16_h, os_h, odn_h,
             sidx, didx, exb, dnb, S, DN, sg0, sg1, sg2, sg3, ss0, ss1):
        cid = lax.axis_index("c")
        sub = lax.axis_index("s")
        tid = cid * 16 + sub
        lanes = jnp.arange(16, dtype=jnp.int32)
        bidx = [jnp.bitwise_xor(lanes, sh) for sh in (8, 4, 2, 1)]
        zeros16 = jnp.zeros((16,), F32)
        ones16 = jnp.ones((16,), F32)
        SHORT = SR - LAST

        def per_slice(fn):
            @pl.when(sub < 15)
            def _():
                fn(sub * RPT, RPT)

            @pl.when(sub == 15)
            def _():
                fn(LAST, SHORT)

        pltpu.sync_copy(srci_h.at[tid], sidx)
        pltpu.sync_copy(dsti_h.at[tid], didx)
        per_slice(lambda o, s: pltpu.sync_copy(
            z16_h.at[pl.ds(0, s)], DN.at[pl.ds(o, s)]))
        plsc.subcore_barrier()

        NSUB = C // CA

        def stage_a(gaA, gzA, gaB, gzB):
            bufs = ((gaA, gzA, sg0, sg1), (gaB, gzB, sg2, sg3))

            def fire(jc, half, bi):
                ga_, gz_, s0, s1 = bufs[bi]
                off = half * CA
                pltpu.async_copy(ta_h.at[sidx.at[jc, pl.ds(off, CA)]],
                                 ga_, s0)
                pltpu.async_copy(zt_h.at[didx.at[jc, pl.ds(off, CA)]],
                                 gz_, s1)

            def wait(bi):
                ga_, gz_, s0, s1 = bufs[bi]
                pltpu.make_async_copy(ta_h.at[sidx.at[0, pl.ds(0, CA)]],
                                      ga_, s0).wait()
                pltpu.make_async_copy(zt_h.at[didx.at[0, pl.ds(0, CA)]],
                                      gz_, s1).wait()

            def compute(jc, half, bi):
                ga_, gz_ = bufs[bi][0], bufs[bi][1]

                def group(g):
                    exv = zeros16
                    for m in range(8):
                        e = g * 8 + m
                        zrow = [gz_[e, pl.ds(i * 16, 16)] for i in range(16)]
                        p0s = [ga_[e, pl.ds(i * 16, 16)] * zrow[i]
                               for i in range(16)]
                        p1s = [ga_[e, pl.ds(256 + i * 16, 16)] * zrow[i]
                               for i in range(16)]
                        while len(p0s) > 1:
                            p0s = [p0s[i] + p0s[i + 1]
                                   for i in range(0, len(p0s), 2)]
                            p1s = [p1s[i] + p1s[i + 1]
                                   for i in range(0, len(p1s), 2)]
                        acc0 = p0s[0]
                        acc1 = p1s[0]
                        for ix in bidx:
                            acc0 = acc0 + acc0.at[ix].get(
                                mode="promise_in_bounds", unique_indices=True)
                            acc1 = acc1 + acc1.at[ix].get(
                                mode="promise_in_bounds", unique_indices=True)
                        d0 = jnp.minimum(
                            jnp.where(acc0 > 0, acc0, acc0 * 0.01), 60.0)
                        d1 = jnp.minimum(
                            jnp.where(acc1 > 0, acc1, acc1 * 0.01), 60.0)
                        e0 = jnp.exp(d0)
                        e1 = jnp.exp(d1)
                        exv = jnp.where(lanes == 2 * m, e0, exv)
                        exv = jnp.where(lanes == 2 * m + 1, e1, exv)
                        dnr = jnp.where(lanes == 0, e0,
                                        jnp.where(lanes == 1, e1,
                                                  jnp.where(lanes == 2,
                                                            ones16, zeros16)))
                        dnb[half * CA + e, :] = dnr
                    exb[jc * (C // 8) + half * (CA // 8) + g, :] = exv
                plsc.parallel_loop(0, CA // 8)(group)

            fire(0, 0, 0)
            TS = NCH * NSUB

            def sub_a(t, bi):
                jc = t // NSUB
                half = lax.rem(t, NSUB)
                wait(bi)
                nt = t + 1

                @pl.when(nt < TS)
                def _():
                    fire(nt // NSUB, lax.rem(nt, NSUB), 1 - bi)
                compute(jc, half, bi)

                @pl.when(half == NSUB - 1)
                def _():
                    pltpu.sync_copy(dnb, DN.at[didx.at[jc]], add=True)

            def pair_a(i, carry):
                sub_a(2 * i, 0)
                sub_a(2 * i + 1, 1)
                return carry
            lax.fori_loop(0, TS // 2, pair_a, 0)

        pl.run_scoped(stage_a,
                      pltpu.VMEM((CA, 512), F32), pltpu.VMEM((CA, 256), F32),
                      pltpu.VMEM((CA, 512), F32), pltpu.VMEM((CA, 256), F32))

        plsc.subcore_barrier()
        per_slice(lambda o, s: pltpu.sync_copy(
            DN.at[pl.ds(o, s)], odn_h.at[cid, pl.ds(o, s)]))

        pts = (p0_h, p1_h, p2_h, p3_h, p4_h, p5_h)
        qts = (q0_h, q1_h, q2_h, q3_h, q4_h, q5_h)

        def stage_b(pjA, qjA, pjB, qjB, pr):
            bufs = ((pjA, qjA, sg0, sg1), (pjB, qjB, sg2, sg3))

            for j in range(NJ):
                pt = pts[j]
                qt = qts[j]
                per_slice(lambda o, s: pltpu.sync_copy(
                    z96_h.at[pl.ds(0, s)], S.at[pl.ds(o, s)]))
                plsc.subcore_barrier()

                def fire(jc, bi):
                    pj_, qj_, s0, s1 = bufs[bi]
                    pltpu.async_copy(pt.at[sidx.at[jc]], pj_, s0)
                    pltpu.async_copy(qt.at[didx.at[jc]], qj_, s1)

                def wait(bi):
                    pj_, qj_, s0, s1 = bufs[bi]
                    pltpu.make_async_copy(pt.at[sidx.at[0]], pj_, s0).wait()
                    pltpu.make_async_copy(qt.at[didx.at[0]], qj_, s1).wait()

                CH = C // 2
                sss = (ss0, ss1)

                def half_b(jc, bi):
                    pj_, qj_ = bufs[bi][0], bufs[bi][1]
                    wait(bi)

                    @pl.when(jc + 1 < NCH)
                    def _():
                        fire(jc + 1, 1 - bi)

                    for h in range(2):
                        rows = pl.ds(h * CH, CH)
                        irows = didx.at[jc, pl.ds(h * CH, CH)]

                        @pl.when(jc > 0)
                        def _():
                            pltpu.make_async_copy(
                                pr.at[rows], S.at[irows], sss[h]).wait()

                        def group(g):
                            gg = h * (CH // 8) + g
                            exv = exb[jc * (C // 8) + gg, :]
                            for m in range(8):
                                e = gg * 8 + m
                                e0 = exv[2 * m]
                                e1 = exv[2 * m + 1]
                                for i in range(W // 16):
                                    hcol = pl.ds(i * 16, 16)
                                    hv = jnp.maximum(
                                        pj_[e, hcol] + qj_[e, hcol], 0.0)
                                    pr[e, hcol] = hv * e0
                                    pr[e, pl.ds(W + i * 16, 16)] = hv * e1
                        plsc.parallel_loop(0, CH // 8)(group)
                        pltpu.async_copy(pr.at[rows], S.at[irows], sss[h],
                                         add=True)

                fire(0, 0)

                def chunk_b(jc2, carry):
                    half_b(2 * jc2, 0)
                    half_b(2 * jc2 + 1, 1)
                    return carry
                lax.fori_loop(0, NCH // 2, chunk_b, 0)

                pltpu.make_async_copy(
                    pr.at[pl.ds(0, CH)],
                    S.at[didx.at[0, pl.ds(0, CH)]], ss0).wait()
                pltpu.make_async_copy(
                    pr.at[pl.ds(CH, CH)],
                    S.at[didx.at[0, pl.ds(CH, CH)]], ss1).wait()
                plsc.subcore_barrier()
                per_slice(lambda o, s: pltpu.sync_copy(
                    S.at[pl.ds(o, s)], os_h.at[cid, j, pl.ds(o, s)]))
                plsc.subcore_barrier()

        pl.run_scoped(stage_b,
                      pltpu.VMEM((C, W), F32), pltpu.VMEM((C, W), F32),
                      pltpu.VMEM((C, W), F32), pltpu.VMEM((C, W), F32),
                      pltpu.VMEM((C, 2 * W), F32))

    return body(srci, dsti, ta, zt, *pqs, z96, z16)


def _post_body(os_ref, dn_ref, w2_ref, f1a_ref, f1b_ref, out_ref):
    dn = dn_ref[0]
    deg = jnp.maximum(dn[0, 2] + dn[1, 2], 1.0)
    ts = []
    for k in range(2):
        cols = []
        for j, (c0, cw) in enumerate(JCOL):
            cols.append((os_ref[0, j, :, k * W:k * W + cw]
                         + os_ref[1, j, :, k * W:k * W + cw]))
        s = jnp.concatenate(cols, axis=1)
        denom = jnp.maximum(dn[0, k] + dn[1, k], 1e-30)
        s = s * (1.0 / (denom * deg))[:, None]
        ts.append(jnp.dot(s, w2_ref[...].T, precision=HI,
                          preferred_element_type=F32))
    cat = jnp.concatenate(ts, axis=1)
    hid = jnp.maximum(jnp.dot(cat, f1a_ref[...].T, precision=HI,
                              preferred_element_type=F32), 0.0)
    out_ref[...] = jnp.dot(hid, f1b_ref[...].T, precision=HI,
                           preferred_element_type=F32)


def _post(os_, dn_t, F2_w2, F1_w1, F1_w2):
    grid = NP // RB
    return pl.pallas_call(
        _post_body,
        grid=(grid,),
        in_specs=[
            pl.BlockSpec((2, NJ, RB, 2 * W), lambda i: (0, 0, i, 0)),
            pl.BlockSpec((1, 2, 16, RB), lambda i: (i, 0, 0, 0)),
            pl.BlockSpec(F2_w2.shape, lambda i: (0, 0)),
            pl.BlockSpec(F1_w1.shape, lambda i: (0, 0)),
            pl.BlockSpec(F1_w2.shape, lambda i: (0, 0)),
        ],
        out_specs=pl.BlockSpec((RB, 256), lambda i: (i, 0)),
        out_shape=jax.ShapeDtypeStruct((NP, 256), F32),
    )(os_, dn_t, F2_w2, F1_w1, F1_w2)


def kernel(zG, xt_enc, edge_index, Ws, Wd, F1_w1, F1_w2, F2_w1, F2_w2):
    n, z = zG.shape
    e = edge_index.shape[1]
    ept = e // NT
    pad = NCH * C - ept
    dummy = n + 1

    src = edge_index[0].astype(jnp.int32).reshape(NT, ept)
    dst = edge_index[1].astype(jnp.int32).reshape(NT, ept)
    srci = jnp.pad(src, ((0, 0), (0, pad)),
                   constant_values=dummy).reshape(NT, NCH, C)
    dsti = jnp.pad(dst, ((0, 0), (0, pad)),
                   constant_values=dummy).reshape(NT, NCH, C)

    zGp = jnp.pad(zG, ((0, NP - n), (0, 0)))
    xtp = jnp.pad(xt_enc, ((0, NP - n), (0, 0)))
    z96 = jnp.zeros((RPT, 2 * W), F32)
    z16 = jnp.zeros((RPT, 16), F32)

    ta, zt, *pqs = _build_tables(zGp, xtp, Ws, Wd, F2_w1)
    os_, odn = _sc_edges(srci, dsti, ta, zt, pqs, z96, z16)
    dn_t = jnp.transpose(odn, (0, 2, 1)).reshape(2, 16, NP // RB, RB)
    dn_t = jnp.transpose(dn_t, (2, 0, 1, 3))
    out = _post(os_, dn_t, F2_w2, F1_w1, F1_w2)
    return out[:n]


# speedup vs baseline: 1.0119x; 1.0119x over previous
import functools

import jax
import jax.numpy as jnp
from jax import lax
from jax.experimental import pallas as pl
from jax.experimental.pallas import tpu as pltpu
from jax.experimental.pallas import tpu_sc as plsc

F32 = jnp.float32
HI = jax.lax.Precision.HIGHEST

NP = 10112
RPT = NP // 16
SR = 10016
LAST = 15 * RPT
NT = 32
C = 128
CA = 16
NCH = 40
W = 48
NJ = 6
RB = 1264
JCOL = [(0, 48), (48, 48), (96, 48), (144, 48), (192, 48), (240, 16)]


def _tables_body(z_ref, x_ref, ws_ref, wd_ref, w1_ref, ta_ref, zt_ref,
                 *pq_refs):
    zb = z_ref[...]
    xb = x_ref[...]
    h = ws_ref.shape[1]
    for k in range(2):
        a = jnp.dot(zb, ws_ref[k].T, precision=HI, preferred_element_type=F32)
        u = jnp.dot(a, wd_ref[k], precision=HI, preferred_element_type=F32)
        ta_ref[:, k * 256:(k + 1) * 256] = u
    zt_ref[...] = zb
    p = jnp.dot(xb, w1_ref[:, :h].T, precision=HI, preferred_element_type=F32)
    q = jnp.dot(xb, w1_ref[:, h:].T, precision=HI, preferred_element_type=F32)
    zpad = jnp.zeros((zb.shape[0], W - JCOL[-1][1]), F32)
    for j, (c0, cw) in enumerate(JCOL):
        ps = p[:, c0:c0 + cw]
        qs = q[:, c0:c0 + cw]
        if cw < W:
            ps = jnp.concatenate([ps, zpad], axis=1)
            qs = jnp.concatenate([qs, zpad], axis=1)
        pq_refs[j][...] = ps
        pq_refs[NJ + j][...] = qs


def _build_tables(zGp, xtp, Ws, Wd, F2_w1):
    grid = NP // RB
    wspec = pl.BlockSpec(Ws.shape, lambda i: (0, 0, 0))
    w1spec = pl.BlockSpec(F2_w1.shape, lambda i: (0, 0))
    row = lambda w: pl.BlockSpec((RB, w), lambda i: (i, 0))
    return pl.pallas_call(
        _tables_body,
        grid=(grid,),
        in_specs=[row(256), row(256), wspec, wspec, w1spec],
        out_specs=[row(512), row(256)] + [row(W)] * (2 * NJ),
        out_shape=[jax.ShapeDtypeStruct((NP, 512), F32),
                   jax.ShapeDtypeStruct((NP, 256), F32)]
                  + [jax.ShapeDtypeStruct((NP, W), F32)] * (2 * NJ),
    )(zGp, xtp, Ws, Wd, F2_w1)


def _sc_edges(srci, dsti, ta, zt, pqs, z96, z16):
    mesh = plsc.VectorSubcoreMesh(core_axis_name="c", subcore_axis_name="s",
                                  num_cores=2, num_subcores=16)

    @functools.partial(
        pl.kernel,
        out_type=(jax.ShapeDtypeStruct((2, NJ, NP, 2 * W), F32),
                  jax.ShapeDtypeStruct((2, NP, 16), F32)),
        mesh=mesh,
        compiler_params=pltpu.CompilerParams(use_tc_tiling_on_sc=False),
        scratch_types=dict(
            sidx=pltpu.VMEM((NCH, C), jnp.int32),
            didx=pltpu.VMEM((NCH, C), jnp.int32),
            exb=pltpu.VMEM((NCH * C // 8, 16), F32),
            dnb=pltpu.VMEM((C, 16), F32),
            S=pltpu.VMEM_SHARED((SR, 2 * W), F32),
            DN=pltpu.VMEM_SHARED((SR, 16), F32),
            sg0=pltpu.SemaphoreType.DMA,
            sg1=pltpu.SemaphoreType.DMA,
            sg2=pltpu.SemaphoreType.DMA,
            sg3=pltpu.SemaphoreType.DMA,
        ),
    )
    def body(srci_h, dsti_h, ta_h, zt_h, p0_h, p1_h, p2_h, p3_h, p4_h, p5_h,
             q0_h, q1_h, q2_h, q3_h, q4_h, q5_h, z96_h, z16_h, os_h, odn_h,
             sidx, didx, exb, dnb, S, DN, sg0, sg1, sg2, sg3):
        cid = lax.axis_index("c")
        sub = lax.axis_index("s")
        tid = cid * 16 + sub
        lanes = jnp.arange(16, dtype=jnp.int32)
        bidx = [jnp.bitwise_xor(lanes, sh) for sh in (8, 4, 2, 1)]
        zeros16 = jnp.zeros((16,), F32)
        ones16 = jnp.ones((16,), F32)
        SHORT = SR - LAST

        def per_slice(fn):
            @pl.when(sub < 15)
            def _():
                fn(sub * RPT, RPT)

            @pl.when(sub == 15)
            def _():
                fn(LAST, SHORT)

        pltpu.sync_copy(srci_h.at[tid], sidx)
        pltpu.sync_copy(dsti_h.at[tid], didx)
        per_slice(lambda o, s: pltpu.sync_copy(
            z16_h.at[pl.ds(0, s)], DN.at[pl.ds(o, s)]))
        plsc.subcore_barrier()

        NSUB = C // CA

        def stage_a(gaA, gzA, gaB, gzB):
            bufs = ((gaA, gzA, sg0, sg1), (gaB, gzB, sg2, sg3))

            def fire(jc, half, bi):
                ga_, gz_, s0, s1 = bufs[bi]
                off = half * CA
                pltpu.async_copy(ta_h.at[sidx.at[jc, pl.ds(off, CA)]],
                                 ga_, s0)
                pltpu.async_copy(zt_h.at[didx.at[jc, pl.ds(off, CA)]],
                                 gz_, s1)

            def wait(bi):
                ga_, gz_, s0, s1 = bufs[bi]
                pltpu.make_async_copy(ta_h.at[sidx.at[0, pl.ds(0, CA)]],
                                      ga_, s0).wait()
                pltpu.make_async_copy(zt_h.at[didx.at[0, pl.ds(0, CA)]],
                                      gz_, s1).wait()

            def compute(jc, half, bi):
                ga_, gz_ = bufs[bi][0], bufs[bi][1]

                def group(g):
                    exv = zeros16
                    for m in range(8):
                        e = g * 8 + m
                        zrow = [gz_[e, pl.ds(i * 16, 16)] for i in range(16)]
                        p0s = [ga_[e, pl.ds(i * 16, 16)] * zrow[i]
                               for i in range(16)]
                        p1s = [ga_[e, pl.ds(256 + i * 16, 16)] * zrow[i]
                               for i in range(16)]
                        while len(p0s) > 1:
                            p0s = [p0s[i] + p0s[i + 1]
                                   for i in range(0, len(p0s), 2)]
                            p1s = [p1s[i] + p1s[i + 1]
                                   for i in range(0, len(p1s), 2)]
                        acc0 = p0s[0]
                        acc1 = p1s[0]
                        for ix in bidx:
                            acc0 = acc0 + acc0.at[ix].get(
                                mode="promise_in_bounds", unique_indices=True)
                            acc1 = acc1 + acc1.at[ix].get(
                                mode="promise_in_bounds", unique_indices=True)
                        d0 = jnp.minimum(
                            jnp.where(acc0 > 0, acc0, acc0 * 0.01), 60.0)
                        d1 = jnp.minimum(
                            jnp.where(acc1 > 0, acc1, acc1 * 0.01), 60.0)
                        e0 = jnp.exp(d0)
                        e1 = jnp.exp(d1)
                        exv = jnp.where(lanes == 2 * m, e0, exv)
                        exv = jnp.where(lanes == 2 * m + 1, e1, exv)
                        dnr = jnp.where(lanes == 0, e0,
                                        jnp.where(lanes == 1, e1,
                                                  jnp.where(lanes == 2,
                                                            ones16, zeros16)))
                        dnb[half * CA + e, :] = dnr
                    exb[jc * (C // 8) + half * (CA // 8) + g, :] = exv
                plsc.parallel_loop(0, CA // 8)(group)

            fire(0, 0, 0)
            TS = NCH * NSUB

            def sub_a(t, bi):
                jc = t // NSUB
                half = lax.rem(t, NSUB)
                wait(bi)
                nt = t + 1

                @pl.when(nt < TS)
                def _():
                    fire(nt // NSUB, lax.rem(nt, NSUB), 1 - bi)
                compute(jc, half, bi)

                @pl.when(half == NSUB - 1)
                def _():
                    pltpu.sync_copy(dnb, DN.at[didx.at[jc]], add=True)

            def pair_a(i, carry):
                sub_a(2 * i, 0)
                sub_a(2 * i + 1, 1)
                return carry
            lax.fori_loop(0, TS // 2, pair_a, 0)

        pl.run_scoped(stage_a,
                      pltpu.VMEM((CA, 512), F32), pltpu.VMEM((CA, 256), F32),
                      pltpu.VMEM((CA, 512), F32), pltpu.VMEM((CA, 256), F32))

        plsc.subcore_barrier()
        per_slice(lambda o, s: pltpu.sync_copy(
            DN.at[pl.ds(o, s)], odn_h.at[cid, pl.ds(o, s)]))

        pts = (p0_h, p1_h, p2_h, p3_h, p4_h, p5_h)
        qts = (q0_h, q1_h, q2_h, q3_h, q4_h, q5_h)

        def stage_b(pjA, qjA, pjB, qjB, pr):
            bufs = ((pjA, qjA, sg0, sg1), (pjB, qjB, sg2, sg3))

            for j in range(NJ):
                pt = pts[j]
                qt = qts[j]
                per_slice(lambda o, s: pltpu.sync_copy(
                    z96_h.at[pl.ds(0, s)], S.at[pl.ds(o, s)]))
                plsc.subcore_barrier()

                def fire(jc, bi):
                    pj_, qj_, s0, s1 = bufs[bi]
                    pltpu.async_copy(pt.at[sidx.at[jc]], pj_, s0)
                    pltpu.async_copy(qt.at[didx.at[jc]], qj_, s1)

                def wait(bi):
                    pj_, qj_, s0, s1 = bufs[bi]
                    pltpu.make_async_copy(pt.at[sidx.at[0]], pj_, s0).wait()
                    pltpu.make_async_copy(qt.at[didx.at[0]], qj_, s1).wait()

                def half_b(jc, bi):
                    pj_, qj_ = bufs[bi][0], bufs[bi][1]
                    wait(bi)

                    @pl.when(jc + 1 < NCH)
                    def _():
                        fire(jc + 1, 1 - bi)

                    def group(g):
                        exv = exb[jc * (C // 8) + g, :]
                        for m in range(8):
                            e = g * 8 + m
                            e0 = exv[2 * m]
                            e1 = exv[2 * m + 1]
                            for i in range(W // 16):
                                hcol = pl.ds(i * 16, 16)
                                hv = jnp.maximum(
                                    pj_[e, hcol] + qj_[e, hcol], 0.0)
                                pr[e, hcol] = hv * e0
                                pr[e, pl.ds(W + i * 16, 16)] = hv * e1
                    plsc.parallel_loop(0, C // 8)(group)
                    pltpu.sync_copy(pr, S.at[didx.at[jc]], add=True)

                fire(0, 0)

                def chunk_b(jc2, carry):
                    half_b(2 * jc2, 0)
                    half_b(2 * jc2 + 1, 1)
                    return carry
                lax.fori_loop(0, NCH // 2, chunk_b, 0)

                plsc.subcore_barrier()
                per_slice(lambda o, s: pltpu.sync_copy(
                    S.at[pl.ds(o, s)], os_h.at[cid, j, pl.ds(o, s)]))
                plsc.subcore_barrier()

        pl.run_scoped(stage_b,
                      pltpu.VMEM((C, W), F32), pltpu.VMEM((C, W), F32),
                      pltpu.VMEM((C, W), F32), pltpu.VMEM((C, W), F32),
                      pltpu.VMEM((C, 2 * W), F32))

    return body(srci, dsti, ta, zt, *pqs, z96, z16)


def _post_body(os_ref, dn_ref, w2_ref, f1a_ref, f1b_ref, out_ref):
    dn = dn_ref[0]
    deg = jnp.maximum(dn[0, 2] + dn[1, 2], 1.0)
    ts = []
    for k in range(2):
        cols = []
        for j, (c0, cw) in enumerate(JCOL):
            cols.append((os_ref[0, j, :, k * W:k * W + cw]
                         + os_ref[1, j, :, k * W:k * W + cw]))
        s = jnp.concatenate(cols, axis=1)
        denom = jnp.maximum(dn[0, k] + dn[1, k], 1e-30)
        s = s * (1.0 / (denom * deg))[:, None]
        ts.append(jnp.dot(s, w2_ref[...].T, precision=HI,
                          preferred_element_type=F32))
    cat = jnp.concatenate(ts, axis=1)
    hid = jnp.maximum(jnp.dot(cat, f1a_ref[...].T, precision=HI,
                              preferred_element_type=F32), 0.0)
    out_ref[...] = jnp.dot(hid, f1b_ref[...].T, precision=HI,
                           preferred_element_type=F32)


def _post(os_, dn_t, F2_w2, F1_w1, F1_w2):
    grid = NP // RB
    return pl.pallas_call(
        _post_body,
        grid=(grid,),
        in_specs=[
            pl.BlockSpec((2, NJ, RB, 2 * W), lambda i: (0, 0, i, 0)),
            pl.BlockSpec((1, 2, 16, RB), lambda i: (i, 0, 0, 0)),
            pl.BlockSpec(F2_w2.shape, lambda i: (0, 0)),
            pl.BlockSpec(F1_w1.shape, lambda i: (0, 0)),
            pl.BlockSpec(F1_w2.shape, lambda i: (0, 0)),
        ],
        out_specs=pl.BlockSpec((RB, 256), lambda i: (i, 0)),
        out_shape=jax.ShapeDtypeStruct((NP, 256), F32),
    )(os_, dn_t, F2_w2, F1_w1, F1_w2)


def kernel(zG, xt_enc, edge_index, Ws, Wd, F1_w1, F1_w2, F2_w1, F2_w2):
    n, z = zG.shape
    e = edge_index.shape[1]
    ept = e // NT
    pad = NCH * C - ept
    dummy = n + 1

    src = edge_index[0].astype(jnp.int32).reshape(NT, ept)
    dst = edge_index[1].astype(jnp.int32).reshape(NT, ept)
    srci = jnp.pad(src, ((0, 0), (0, pad)),
                   constant_values=dummy).reshape(NT, NCH, C)
    dsti = jnp.pad(dst, ((0, 0), (0, pad)),
                   constant_values=dummy).reshape(NT, NCH, C)

    zGp = jnp.pad(zG, ((0, NP - n), (0, 0)))
    xtp = jnp.pad(xt_enc, ((0, NP - n), (0, 0)))
    z96 = jnp.zeros((RPT, 2 * W), F32)
    z16 = jnp.zeros((RPT, 16), F32)

    ta, zt, *pqs = _build_tables(zGp, xtp, Ws, Wd, F2_w1)
    os_, odn = _sc_edges(srci, dsti, ta, zt, pqs, z96, z16)
    dn_t = jnp.transpose(odn, (0, 2, 1)).reshape(2, 16, NP // RB, RB)
    dn_t = jnp.transpose(dn_t, (2, 0, 1, 3))
    out = _post(os_, dn_t, F2_w2, F1_w1, F1_w2)
    return out[:n]
